# Initial kernel scaffold; baseline (speedup 1.0000x reference)
import jax
import jax.numpy as jnp
from jax.experimental import pallas as pl


def kernel(x, edge_index, W_self1, b_self1, W_neigh1, b_neigh1, gamma1, beta1, W_self2, b_self2, W_neigh2, b_neigh2, gamma2, beta2, W_head, b_head):
    raise NotImplementedError("write your pallas kernel here")


# speedup vs baseline: 14.8119x; 14.8119x over previous
import functools

import jax
import jax.numpy as jnp
from jax import lax
from jax.experimental import pallas as pl
from jax.experimental.pallas import tpu as pltpu
from jax.experimental.pallas import tpu_sc as plsc

N = 10000
NP = 10240
E = 320000
F = 128
FH = F // 2
OUT = 64
EPS = 1e-5

ROW = 80
NCORES = 2
NSUB = 16
CROWS = 5
NCHUNK = E // (ROW * CROWS * NSUB)
CE = CROWS * ROW
RT = NP // NSUB

_SC_PARAMS = pltpu.CompilerParams(use_tc_tiling_on_sc=False)


def _agg_body(with_deg, *refs):
    if with_deg:
        (x_hbm, ei_hbm, p_hbm, degp_hbm,
         accum, sidx, didx, rows, rows2, sg1, ss1, sg2, ss2,
         degacc, zbuf, ones, sdg) = refs
    else:
        (x_hbm, ei_hbm, p_hbm,
         accum, sidx, didx, rows, rows2, sg1, ss1, sg2, ss2) = refs

    cid = lax.axis_index("c")
    sid = lax.axis_index("s")

    pltpu.sync_copy(x_hbm.at[cid, pl.ds(sid * RT, RT)],
                    accum.at[pl.ds(sid * RT, RT)])
    if with_deg:
        for i in range(RT // 16):
            zbuf[pl.ds(i * 16, 16)] = jnp.zeros((16,), jnp.float32)
        for i in range(ROW // 16):
            ones[pl.ds(i * 16, 16)] = jnp.full((16,), 1.0, jnp.float32)
        pltpu.sync_copy(zbuf, degacc.at[pl.ds(sid * RT, RT)])
    plsc.subcore_barrier()

    def chunk(i, carry):
        pltpu.sync_copy(ei_hbm.at[cid, sid, i], sidx)
        pltpu.sync_copy(ei_hbm.at[1 - cid, sid, i], didx)
        g1 = [pltpu.async_copy(x_hbm.at[cid].at[sidx.at[j]],
                               rows.at[pl.ds(j * ROW, ROW)], sg1)
              for j in range(CROWS)]
        if with_deg:
            d1 = [pltpu.async_copy(ones, degacc.at[didx.at[j]], sdg, add=True)
                  for j in range(CROWS)]
        for h in g1:
            h.wait()
        s1 = [pltpu.async_copy(rows.at[pl.ds(j * ROW, ROW)],
                               accum.at[didx.at[j]], ss1, add=True)
              for j in range(CROWS)]
        g2 = [pltpu.async_copy(x_hbm.at[cid].at[didx.at[j]],
                               rows2.at[pl.ds(j * ROW, ROW)], sg2)
              for j in range(CROWS)]
        for h in g2:
            h.wait()
        s2 = [pltpu.async_copy(rows2.at[pl.ds(j * ROW, ROW)],
                               accum.at[sidx.at[j]], ss2, add=True)
              for j in range(CROWS)]
        for h in s1 + s2:
            h.wait()
        if with_deg:
            for h in d1:
                h.wait()
        return carry

    lax.fori_loop(0, NCHUNK, chunk, 0)
    plsc.subcore_barrier()

    pltpu.sync_copy(accum.at[pl.ds(sid * RT, RT)],
                    p_hbm.at[cid, pl.ds(sid * RT, RT)])
    if with_deg:
        pltpu.sync_copy(degacc.at[pl.ds(sid * RT, RT)],
                        degp_hbm.at[cid, sid, 0])


def _make_agg(with_deg):
    mesh = plsc.VectorSubcoreMesh(core_axis_name="c", subcore_axis_name="s",
                                  num_cores=NCORES, num_subcores=NSUB)
    out_type = [jax.ShapeDtypeStruct((NCORES, NP, FH), jnp.float32)]
    scratch = [
        pltpu.VMEM_SHARED((NP, FH), jnp.float32),
        pltpu.VMEM((CROWS, ROW), jnp.int32),
        pltpu.VMEM((CROWS, ROW), jnp.int32),
        pltpu.VMEM((CE, FH), jnp.float32),
        pltpu.VMEM((CE, FH), jnp.float32),
        pltpu.SemaphoreType.DMA,
        pltpu.SemaphoreType.DMA,
        pltpu.SemaphoreType.DMA,
        pltpu.SemaphoreType.DMA,
    ]
    if with_deg:
        out_type.append(
            jax.ShapeDtypeStruct((NCORES, NSUB, 1, RT), jnp.float32))
        scratch += [
            pltpu.VMEM_SHARED((NP,), jnp.float32),
            pltpu.VMEM((RT,), jnp.float32),
            pltpu.VMEM((ROW,), jnp.float32),
            pltpu.SemaphoreType.DMA,
        ]
    return pl.kernel(
        functools.partial(_agg_body, with_deg),
        out_type=out_type,
        mesh=mesh,
        scratch_types=scratch,
        compiler_params=_SC_PARAMS,
    )


_agg_with_deg = _make_agg(True)
_agg_no_deg = _make_agg(False)


def _dot_t(a, w):
    return lax.dot_general(a, w, (((1,), (1,)), ((), ())),
                           preferred_element_type=jnp.float32)


def _bn_relu(t, gamma, beta):
    mask = lax.broadcasted_iota(jnp.int32, (NP, 1), 0) < N
    m = jnp.sum(jnp.where(mask, t, 0.0), axis=0, keepdims=True) / N
    d = jnp.where(mask, t - m, 0.0)
    v = jnp.sum(d * d, axis=0, keepdims=True) / N
    h = (t - m) * lax.rsqrt(v + EPS) * gamma[None, :] + beta[None, :]
    return jnp.maximum(h, 0.0)


def _layer1_body(x2_ref, p_ref, dp_ref, ws_ref, bs_ref, wn_ref, bn_ref,
                 g_ref, b_ref, h1s_ref):
    x = jnp.concatenate([x2_ref[0], x2_ref[1]], axis=1)
    agg = jnp.concatenate([p_ref[0], p_ref[1]], axis=1)
    deg = dp_ref[:, 0:1] + dp_ref[:, 1:2] + 1.0
    t = (_dot_t(x, ws_ref[...]) + _dot_t(agg / deg, wn_ref[...])
         + (bs_ref[...] + bn_ref[...])[None, :])
    h1 = _bn_relu(t, g_ref[...], b_ref[...])
    h1s_ref[0] = h1[:, :FH]
    h1s_ref[1] = h1[:, FH:]


def _layer2_body(h1s_ref, p_ref, dp_ref, ws_ref, bs_ref, wn_ref, bn_ref,
                 g_ref, b_ref, wh_ref, bh_ref, out_ref):
    h1 = jnp.concatenate([h1s_ref[0], h1s_ref[1]], axis=1)
    agg = jnp.concatenate([p_ref[0], p_ref[1]], axis=1)
    deg = dp_ref[:, 0:1] + dp_ref[:, 1:2] + 1.0
    t = (_dot_t(h1, ws_ref[...]) + _dot_t(agg / deg, wn_ref[...])
         + (bs_ref[...] + bn_ref[...])[None, :])
    h2 = _bn_relu(t, g_ref[...], b_ref[...])
    out_ref[...] = _dot_t(h2[:N], wh_ref[...]) + bh_ref[...][None, :]


_layer1 = pl.pallas_call(
    _layer1_body, out_shape=jax.ShapeDtypeStruct((NCORES, NP, FH),
                                                 jnp.float32))
_layer2 = pl.pallas_call(
    _layer2_body, out_shape=jax.ShapeDtypeStruct((N, OUT), jnp.float32))


def kernel(x, edge_index, W_self1, b_self1, W_neigh1, b_neigh1, gamma1, beta1,
           W_self2, b_self2, W_neigh2, b_neigh2, gamma2, beta2, W_head, b_head):
    x_pad = jnp.pad(x, ((0, NP - N), (0, 0)))
    x2 = jnp.stack([x_pad[:, :FH], x_pad[:, FH:]])
    ei5 = edge_index.reshape(2, NSUB, NCHUNK, CROWS, ROW)
    p1, degp = _agg_with_deg(x2, ei5)
    dp_t = degp.reshape(NCORES, NP).T
    h1s = _layer1(x2, p1, dp_t, W_self1, b_self1, W_neigh1, b_neigh1,
                  gamma1, beta1)
    (p2,) = _agg_no_deg(h1s, ei5)
    return _layer2(h1s, p2, dp_t, W_self2, b_self2, W_neigh2, b_neigh2,
                   gamma2, beta2, W_head, b_head)


# speedup vs baseline: 19.7575x; 1.3339x over previous
import functools

import jax
import jax.numpy as jnp
from jax import lax
from jax.experimental import pallas as pl
from jax.experimental.pallas import tpu as pltpu
from jax.experimental.pallas import tpu_sc as plsc

N = 10000
NP = 10240
E = 320000
F = 128
FH = F // 2
OUT = 64
EPS = 1e-5

ROW = 80
NCORES = 2
NSUB = 16
NCH = E // (ROW * NSUB)
RT = NP // NSUB

_SC_PARAMS = pltpu.CompilerParams(use_tc_tiling_on_sc=False)


def _agg_body(with_deg, *refs):
    if with_deg:
        (x_hbm, ei_hbm, p_hbm, degp_hbm,
         accum, sidx, didx, rows, rows2,
         sg1a, sg1b, sg2a, sg2b, ss1a, ss1b, ss2a, ss2b,
         degacc, zbuf, ones, sdg) = refs
    else:
        (x_hbm, ei_hbm, p_hbm,
         accum, sidx, didx, rows, rows2,
         sg1a, sg1b, sg2a, sg2b, ss1a, ss1b, ss2a, ss2b) = refs
    sg1 = (sg1a, sg1b)
    sg2 = (sg2a, sg2b)
    ss1 = (ss1a, ss1b)
    ss2 = (ss2a, ss2b)

    cid = lax.axis_index("c")
    sid = lax.axis_index("s")

    pltpu.sync_copy(x_hbm.at[cid, pl.ds(sid * RT, RT)],
                    accum.at[pl.ds(sid * RT, RT)])
    pltpu.sync_copy(ei_hbm.at[cid, sid], sidx)
    pltpu.sync_copy(ei_hbm.at[1 - cid, sid], didx)
    if with_deg:
        for i in range(RT // 16):
            zbuf[pl.ds(i * 16, 16)] = jnp.zeros((16,), jnp.float32)
        for i in range(ROW // 16):
            ones[pl.ds(i * 16, 16)] = jnp.full((16,), 1.0, jnp.float32)
        pltpu.sync_copy(zbuf, degacc.at[pl.ds(sid * RT, RT)])
    plsc.subcore_barrier()

    def fire_g(i, b):
        pltpu.make_async_copy(x_hbm.at[cid].at[sidx.at[i]],
                              rows.at[b], sg1[b]).start()
        pltpu.make_async_copy(x_hbm.at[cid].at[didx.at[i]],
                              rows2.at[b], sg2[b]).start()
        if with_deg:
            pltpu.make_async_copy(ones, degacc.at[didx.at[i]],
                                  sdg).start(add=True)

    def fire_s(i, b):
        pltpu.make_async_copy(x_hbm.at[cid].at[sidx.at[i]],
                              rows.at[b], sg1[b]).wait()
        pltpu.make_async_copy(rows.at[b], accum.at[didx.at[i]],
                              ss1[b]).start(add=True)
        pltpu.make_async_copy(x_hbm.at[cid].at[didx.at[i]],
                              rows2.at[b], sg2[b]).wait()
        pltpu.make_async_copy(rows2.at[b], accum.at[sidx.at[i]],
                              ss2[b]).start(add=True)

    def wait_s(i, b):
        pltpu.make_async_copy(rows.at[b], accum.at[didx.at[i]],
                              ss1[b]).wait()
        pltpu.make_async_copy(rows2.at[b], accum.at[sidx.at[i]],
                              ss2[b]).wait()
        if with_deg:
            pltpu.make_async_copy(ones, degacc.at[didx.at[i]], sdg).wait()

    fire_g(0, 0)
    fire_g(1, 1)
    fire_s(0, 0)

    def step(p, carry):
        for b in (0, 1):
            i = 2 * p + b
            wait_s(i - 2, b)
            fire_g(i, b)
            fire_s(i - 1, 1 - b)
        return carry

    lax.fori_loop(1, NCH // 2, step, 0)
    fire_s(NCH - 1, 1)
    wait_s(NCH - 2, 0)
    wait_s(NCH - 1, 1)
    plsc.subcore_barrier()

    pltpu.sync_copy(accum.at[pl.ds(sid * RT, RT)],
                    p_hbm.at[cid, pl.ds(sid * RT, RT)])
    if with_deg:
        pltpu.sync_copy(degacc.at[pl.ds(sid * RT, RT)],
                        degp_hbm.at[cid, sid, 0])


def _make_agg(with_deg):
    mesh = plsc.VectorSubcoreMesh(core_axis_name="c", subcore_axis_name="s",
                                  num_cores=NCORES, num_subcores=NSUB)
    out_type = [jax.ShapeDtypeStruct((NCORES, NP, FH), jnp.float32)]
    scratch = [
        pltpu.VMEM_SHARED((NP, FH), jnp.float32),
        pltpu.VMEM((NCH, ROW), jnp.int32),
        pltpu.VMEM((NCH, ROW), jnp.int32),
        pltpu.VMEM((2, ROW, FH), jnp.float32),
        pltpu.VMEM((2, ROW, FH), jnp.float32),
    ] + [pltpu.SemaphoreType.DMA] * 8
    if with_deg:
        out_type.append(
            jax.ShapeDtypeStruct((NCORES, NSUB, 1, RT), jnp.float32))
        scratch += [
            pltpu.VMEM_SHARED((NP,), jnp.float32),
            pltpu.VMEM((RT,), jnp.float32),
            pltpu.VMEM((ROW,), jnp.float32),
            pltpu.SemaphoreType.DMA,
        ]
    return pl.kernel(
        functools.partial(_agg_body, with_deg),
        out_type=out_type,
        mesh=mesh,
        scratch_types=scratch,
        compiler_params=_SC_PARAMS,
    )


_agg_with_deg = _make_agg(True)
_agg_no_deg = _make_agg(False)


def _dot_t(a, w):
    return lax.dot_general(a, w, (((1,), (1,)), ((), ())),
                           preferred_element_type=jnp.float32)


def _bn_relu(t, gamma, beta):
    mask = lax.broadcasted_iota(jnp.int32, (NP, 1), 0) < N
    m = jnp.sum(jnp.where(mask, t, 0.0), axis=0, keepdims=True) / N
    d = jnp.where(mask, t - m, 0.0)
    v = jnp.sum(d * d, axis=0, keepdims=True) / N
    h = (t - m) * lax.rsqrt(v + EPS) * gamma[None, :] + beta[None, :]
    return jnp.maximum(h, 0.0)


def _layer1_body(x2_ref, p_ref, dp_ref, ws_ref, bs_ref, wn_ref, bn_ref,
                 g_ref, b_ref, h1s_ref):
    x = jnp.concatenate([x2_ref[0], x2_ref[1]], axis=1)
    agg = jnp.concatenate([p_ref[0], p_ref[1]], axis=1)
    deg = dp_ref[:, 0:1] + dp_ref[:, 1:2] + 1.0
    t = (_dot_t(x, ws_ref[...]) + _dot_t(agg / deg, wn_ref[...])
         + (bs_ref[...] + bn_ref[...])[None, :])
    h1 = _bn_relu(t, g_ref[...], b_ref[...])
    h1s_ref[0] = h1[:, :FH]
    h1s_ref[1] = h1[:, FH:]


def _layer2_body(h1s_ref, p_ref, dp_ref, ws_ref, bs_ref, wn_ref, bn_ref,
                 g_ref, b_ref, wh_ref, bh_ref, out_ref):
    h1 = jnp.concatenate([h1s_ref[0], h1s_ref[1]], axis=1)
    agg = jnp.concatenate([p_ref[0], p_ref[1]], axis=1)
    deg = dp_ref[:, 0:1] + dp_ref[:, 1:2] + 1.0
    t = (_dot_t(h1, ws_ref[...]) + _dot_t(agg / deg, wn_ref[...])
         + (bs_ref[...] + bn_ref[...])[None, :])
    h2 = _bn_relu(t, g_ref[...], b_ref[...])
    out_ref[...] = _dot_t(h2[:N], wh_ref[...]) + bh_ref[...][None, :]


_layer1 = pl.pallas_call(
    _layer1_body, out_shape=jax.ShapeDtypeStruct((NCORES, NP, FH),
                                                 jnp.float32))
_layer2 = pl.pallas_call(
    _layer2_body, out_shape=jax.ShapeDtypeStruct((N, OUT), jnp.float32))


def kernel(x, edge_index, W_self1, b_self1, W_neigh1, b_neigh1, gamma1, beta1,
           W_self2, b_self2, W_neigh2, b_neigh2, gamma2, beta2, W_head, b_head):
    x_pad = jnp.pad(x, ((0, NP - N), (0, 0)))
    x2 = jnp.stack([x_pad[:, :FH], x_pad[:, FH:]])
    ei4 = edge_index.reshape(2, NSUB, NCH, ROW)
    p1, degp = _agg_with_deg(x2, ei4)
    dp_t = degp.reshape(NCORES, NP).T
    h1s = _layer1(x2, p1, dp_t, W_self1, b_self1, W_neigh1, b_neigh1,
                  gamma1, beta1)
    (p2,) = _agg_no_deg(h1s, ei4)
    return _layer2(h1s, p2, dp_t, W_self2, b_self2, W_neigh2, b_neigh2,
                   gamma2, beta2, W_head, b_head)


# speedup vs baseline: 26.1344x; 1.3228x over previous
import functools

import jax
import jax.numpy as jnp
from jax import lax
from jax.experimental import pallas as pl
from jax.experimental.pallas import tpu as pltpu
from jax.experimental.pallas import tpu_sc as plsc

N = 10000
NP = 10240
E = 320000
F = 128
FH = F // 2
OUT = 64
EPS = 1e-5

ROW = 80
NCORES = 2
NSUB = 16
NCH = E // (ROW * NSUB)
RT = NP // NSUB

_SC_PARAMS = pltpu.CompilerParams(use_tc_tiling_on_sc=False)


def _agg_body(with_deg, *refs):
    if with_deg:
        (x_hbm, ei_hbm, p_hbm, degp_hbm,
         accum, sidx, didx, rows, rows2,
         sg1a, sg1b, sg2a, sg2b, ss1a, ss1b, ss2a, ss2b,
         degacc, zbuf, ones, sdg) = refs
    else:
        (x_hbm, ei_hbm, p_hbm,
         accum, sidx, didx, rows, rows2,
         sg1a, sg1b, sg2a, sg2b, ss1a, ss1b, ss2a, ss2b) = refs
    sg1 = (sg1a, sg1b)
    sg2 = (sg2a, sg2b)
    ss1 = (ss1a, ss1b)
    ss2 = (ss2a, ss2b)

    cid = lax.axis_index("c")
    sid = lax.axis_index("s")

    pltpu.sync_copy(x_hbm.at[cid, pl.ds(sid * RT, RT)],
                    accum.at[pl.ds(sid * RT, RT)])
    pltpu.sync_copy(ei_hbm.at[cid, sid], sidx)
    pltpu.sync_copy(ei_hbm.at[1 - cid, sid], didx)
    if with_deg:
        for i in range(RT // 16):
            zbuf[pl.ds(i * 16, 16)] = jnp.zeros((16,), jnp.float32)
        for i in range(ROW // 16):
            ones[pl.ds(i * 16, 16)] = jnp.full((16,), 1.0, jnp.float32)
        pltpu.sync_copy(zbuf, degacc.at[pl.ds(sid * RT, RT)])
    plsc.subcore_barrier()

    def fire_g(i, b):
        pltpu.make_async_copy(x_hbm.at[cid].at[sidx.at[i]],
                              rows.at[b], sg1[b]).start()
        pltpu.make_async_copy(x_hbm.at[cid].at[didx.at[i]],
                              rows2.at[b], sg2[b]).start()
        if with_deg:
            pltpu.make_async_copy(ones, degacc.at[didx.at[i]],
                                  sdg).start(add=True)

    def fire_s(i, b):
        pltpu.make_async_copy(x_hbm.at[cid].at[sidx.at[i]],
                              rows.at[b], sg1[b]).wait()
        pltpu.make_async_copy(rows.at[b], accum.at[didx.at[i]],
                              ss1[b]).start(add=True)
        pltpu.make_async_copy(x_hbm.at[cid].at[didx.at[i]],
                              rows2.at[b], sg2[b]).wait()
        pltpu.make_async_copy(rows2.at[b], accum.at[sidx.at[i]],
                              ss2[b]).start(add=True)

    def wait_s(i, b):
        pltpu.make_async_copy(rows.at[b], accum.at[didx.at[i]],
                              ss1[b]).wait()
        pltpu.make_async_copy(rows2.at[b], accum.at[sidx.at[i]],
                              ss2[b]).wait()
        if with_deg:
            pltpu.make_async_copy(ones, degacc.at[didx.at[i]], sdg).wait()

    fire_g(0, 0)
    fire_g(1, 1)
    fire_s(0, 0)

    def step(p, carry):
        for b in (0, 1):
            i = 2 * p + b
            wait_s(i - 2, b)
            fire_g(i, b)
            fire_s(i - 1, 1 - b)
        return carry

    lax.fori_loop(1, NCH // 2, step, 0)
    fire_s(NCH - 1, 1)
    wait_s(NCH - 2, 0)
    wait_s(NCH - 1, 1)
    plsc.subcore_barrier()

    pltpu.sync_copy(accum.at[pl.ds(sid * RT, RT)],
                    p_hbm.at[cid, pl.ds(sid * RT, RT)])
    if with_deg:
        pltpu.sync_copy(degacc.at[pl.ds(sid * RT, RT)],
                        degp_hbm.at[cid, sid, 0])


def _make_agg(with_deg):
    mesh = plsc.VectorSubcoreMesh(core_axis_name="c", subcore_axis_name="s",
                                  num_cores=NCORES, num_subcores=NSUB)
    out_type = [jax.ShapeDtypeStruct((NCORES, NP, FH), jnp.bfloat16)]
    scratch = [
        pltpu.VMEM_SHARED((NP, FH), jnp.bfloat16),
        pltpu.VMEM((NCH, ROW), jnp.int32),
        pltpu.VMEM((NCH, ROW), jnp.int32),
        pltpu.VMEM((2, ROW, FH), jnp.bfloat16),
        pltpu.VMEM((2, ROW, FH), jnp.bfloat16),
    ] + [pltpu.SemaphoreType.DMA] * 8
    if with_deg:
        out_type.append(
            jax.ShapeDtypeStruct((NCORES, NSUB, 1, RT), jnp.float32))
        scratch += [
            pltpu.VMEM_SHARED((NP,), jnp.float32),
            pltpu.VMEM((RT,), jnp.float32),
            pltpu.VMEM((ROW,), jnp.float32),
            pltpu.SemaphoreType.DMA,
        ]
    return pl.kernel(
        functools.partial(_agg_body, with_deg),
        out_type=out_type,
        mesh=mesh,
        scratch_types=scratch,
        compiler_params=_SC_PARAMS,
    )


_agg_with_deg = _make_agg(True)
_agg_no_deg = _make_agg(False)


def _dot_t(a, w):
    return lax.dot_general(a, w, (((1,), (1,)), ((), ())),
                           preferred_element_type=jnp.float32)


def _bn_relu(t, gamma, beta):
    mask = lax.broadcasted_iota(jnp.int32, (NP, 1), 0) < N
    m = jnp.sum(jnp.where(mask, t, 0.0), axis=0, keepdims=True) / N
    d = jnp.where(mask, t - m, 0.0)
    v = jnp.sum(d * d, axis=0, keepdims=True) / N
    h = (t - m) * lax.rsqrt(v + EPS) * gamma[None, :] + beta[None, :]
    return jnp.maximum(h, 0.0)


def _layer1_body(x_ref, p_ref, dp_ref, ws_ref, bs_ref, wn_ref, bn_ref,
                 g_ref, b_ref, h1f_ref, h1s_ref):
    x = x_ref[...]
    agg = jnp.concatenate([p_ref[0], p_ref[1]], axis=1).astype(jnp.float32)
    deg = dp_ref[:, 0:1] + dp_ref[:, 1:2] + 1.0
    t = (_dot_t(x, ws_ref[...]) + _dot_t(agg / deg, wn_ref[...])
         + (bs_ref[...] + bn_ref[...])[None, :])
    h1 = _bn_relu(t, g_ref[...], b_ref[...])
    h1f_ref[...] = h1
    h1b = h1.astype(jnp.bfloat16)
    h1s_ref[0] = h1b[:, :FH]
    h1s_ref[1] = h1b[:, FH:]


def _layer2_body(h1f_ref, p_ref, dp_ref, ws_ref, bs_ref, wn_ref, bn_ref,
                 g_ref, b_ref, wh_ref, bh_ref, out_ref):
    h1 = h1f_ref[...]
    agg = jnp.concatenate([p_ref[0], p_ref[1]], axis=1).astype(jnp.float32)
    deg = dp_ref[:, 0:1] + dp_ref[:, 1:2] + 1.0
    t = (_dot_t(h1, ws_ref[...]) + _dot_t(agg / deg, wn_ref[...])
         + (bs_ref[...] + bn_ref[...])[None, :])
    h2 = _bn_relu(t, g_ref[...], b_ref[...])
    out_ref[...] = _dot_t(h2[:N], wh_ref[...]) + bh_ref[...][None, :]


_layer1 = pl.pallas_call(
    _layer1_body, out_shape=[jax.ShapeDtypeStruct((NP, F), jnp.float32),
                             jax.ShapeDtypeStruct((NCORES, NP, FH),
                                                  jnp.bfloat16)])
_layer2 = pl.pallas_call(
    _layer2_body, out_shape=jax.ShapeDtypeStruct((N, OUT), jnp.float32))


def kernel(x, edge_index, W_self1, b_self1, W_neigh1, b_neigh1, gamma1, beta1,
           W_self2, b_self2, W_neigh2, b_neigh2, gamma2, beta2, W_head, b_head):
    x_pad = jnp.pad(x, ((0, NP - N), (0, 0)))
    xb = x_pad.astype(jnp.bfloat16)
    x2 = jnp.stack([xb[:, :FH], xb[:, FH:]])
    ei4 = edge_index.reshape(2, NSUB, NCH, ROW)
    p1, degp = _agg_with_deg(x2, ei4)
    dp_t = degp.reshape(NCORES, NP).T
    h1f, h1s = _layer1(x_pad, p1, dp_t, W_self1, b_self1, W_neigh1,
                       b_neigh1, gamma1, beta1)
    (p2,) = _agg_no_deg(h1s, ei4)
    return _layer2(h1f, p2, dp_t, W_self2, b_self2, W_neigh2, b_neigh2,
                   gamma2, beta2, W_head, b_head)


# speedup vs baseline: 28.0319x; 1.0726x over previous
import functools

import jax
import jax.numpy as jnp
from jax import lax
from jax.experimental import pallas as pl
from jax.experimental.pallas import tpu as pltpu
from jax.experimental.pallas import tpu_sc as plsc

N = 10000
NP = 10240
E = 320000
F = 128
FH = F // 2
OUT = 64
EPS = 1e-5

ROW = 80
NCORES = 2
NSUB = 16
NCH = E // (ROW * NSUB)
NSLOT = 5
RT = NP // NSUB

_SC_PARAMS = pltpu.CompilerParams(use_tc_tiling_on_sc=False)


def _agg_body(with_deg, *refs):
    nin = 4 if with_deg else 3
    x_hbm, ei_hbm, p_hbm = refs[0], refs[1], refs[2]
    degp_hbm = refs[3] if with_deg else None
    accum, sidx, didx, rows, rows2 = refs[nin:nin + 5]
    sems = refs[nin + 5:nin + 5 + 4 * NSLOT]
    sg1 = sems[0:NSLOT]
    sg2 = sems[NSLOT:2 * NSLOT]
    ss1 = sems[2 * NSLOT:3 * NSLOT]
    ss2 = sems[3 * NSLOT:4 * NSLOT]
    if with_deg:
        degacc, zbuf, ones, sdg = refs[nin + 5 + 4 * NSLOT:]

    cid = lax.axis_index("c")
    sid = lax.axis_index("s")

    pltpu.sync_copy(x_hbm.at[cid, pl.ds(sid * RT, RT)],
                    accum.at[pl.ds(sid * RT, RT)])
    pltpu.sync_copy(ei_hbm.at[cid, sid], sidx)
    pltpu.sync_copy(ei_hbm.at[1 - cid, sid], didx)
    if with_deg:
        for i in range(RT // 16):
            zbuf[pl.ds(i * 16, 16)] = jnp.zeros((16,), jnp.float32)
        for i in range(ROW // 16):
            ones[pl.ds(i * 16, 16)] = jnp.full((16,), 1.0, jnp.float32)
        pltpu.sync_copy(zbuf, degacc.at[pl.ds(sid * RT, RT)])
    plsc.subcore_barrier()

    def fire_g(i, b):
        pltpu.make_async_copy(x_hbm.at[cid].at[sidx.at[i]],
                              rows.at[b], sg1[b]).start()
        pltpu.make_async_copy(x_hbm.at[cid].at[didx.at[i]],
                              rows2.at[b], sg2[b]).start()
        if with_deg:
            pltpu.make_async_copy(ones, degacc.at[didx.at[i]],
                                  sdg).start(add=True)

    def fire_s(i, b):
        pltpu.make_async_copy(x_hbm.at[cid].at[sidx.at[i]],
                              rows.at[b], sg1[b]).wait()
        pltpu.make_async_copy(rows.at[b], accum.at[didx.at[i]],
                              ss1[b]).start(add=True)
        pltpu.make_async_copy(x_hbm.at[cid].at[didx.at[i]],
                              rows2.at[b], sg2[b]).wait()
        pltpu.make_async_copy(rows2.at[b], accum.at[sidx.at[i]],
                              ss2[b]).start(add=True)

    def wait_s(i, b):
        pltpu.make_async_copy(rows.at[b], accum.at[didx.at[i]],
                              ss1[b]).wait()
        pltpu.make_async_copy(rows2.at[b], accum.at[sidx.at[i]],
                              ss2[b]).wait()
        if with_deg:
            pltpu.make_async_copy(ones, degacc.at[didx.at[i]], sdg).wait()

    fire_g(0, 0)
    for b in range(1, NSLOT):
        fire_g(b, b)
        fire_s(b - 1, b - 1)

    def step(p, carry):
        for b in range(NSLOT):
            i = NSLOT * p + b
            wait_s(i - NSLOT, b)
            fire_g(i, b)
            fire_s(i - 1, (b - 1) % NSLOT)
        return carry

    lax.fori_loop(1, NCH // NSLOT, step, 0)
    fire_s(NCH - 1, NSLOT - 1)
    for b in range(NSLOT):
        wait_s(NCH - NSLOT + b, b)
    plsc.subcore_barrier()

    pltpu.sync_copy(accum.at[pl.ds(sid * RT, RT)],
                    p_hbm.at[cid, pl.ds(sid * RT, RT)])
    if with_deg:
        pltpu.sync_copy(degacc.at[pl.ds(sid * RT, RT)],
                        degp_hbm.at[cid, sid, 0])


def _make_agg(with_deg):
    mesh = plsc.VectorSubcoreMesh(core_axis_name="c", subcore_axis_name="s",
                                  num_cores=NCORES, num_subcores=NSUB)
    out_type = [jax.ShapeDtypeStruct((NCORES, NP, FH), jnp.bfloat16)]
    scratch = [
        pltpu.VMEM_SHARED((NP, FH), jnp.bfloat16),
        pltpu.VMEM((NCH, ROW), jnp.int32),
        pltpu.VMEM((NCH, ROW), jnp.int32),
        pltpu.VMEM((NSLOT, ROW, FH), jnp.bfloat16),
        pltpu.VMEM((NSLOT, ROW, FH), jnp.bfloat16),
    ] + [pltpu.SemaphoreType.DMA] * (4 * NSLOT)
    if with_deg:
        out_type.append(
            jax.ShapeDtypeStruct((NCORES, NSUB, 1, RT), jnp.float32))
        scratch += [
            pltpu.VMEM_SHARED((NP,), jnp.float32),
            pltpu.VMEM((RT,), jnp.float32),
            pltpu.VMEM((ROW,), jnp.float32),
            pltpu.SemaphoreType.DMA,
        ]
    return pl.kernel(
        functools.partial(_agg_body, with_deg),
        out_type=out_type,
        mesh=mesh,
        scratch_types=scratch,
        compiler_params=_SC_PARAMS,
    )


_agg_with_deg = _make_agg(True)
_agg_no_deg = _make_agg(False)


def _dot_t(a, w):
    return lax.dot_general(a, w, (((1,), (1,)), ((), ())),
                           preferred_element_type=jnp.float32)


def _bn_relu(t, gamma, beta):
    mask = lax.broadcasted_iota(jnp.int32, (NP, 1), 0) < N
    m = jnp.sum(jnp.where(mask, t, 0.0), axis=0, keepdims=True) / N
    d = jnp.where(mask, t - m, 0.0)
    v = jnp.sum(d * d, axis=0, keepdims=True) / N
    h = (t - m) * lax.rsqrt(v + EPS) * gamma[None, :] + beta[None, :]
    return jnp.maximum(h, 0.0)


def _layer1_body(x_ref, p_ref, dp_ref, ws_ref, bs_ref, wn_ref, bn_ref,
                 g_ref, b_ref, h1f_ref, h1s_ref):
    x = x_ref[...]
    agg = jnp.concatenate([p_ref[0], p_ref[1]], axis=1).astype(jnp.float32)
    deg = dp_ref[:, 0:1] + dp_ref[:, 1:2] + 1.0
    t = (_dot_t(x, ws_ref[...]) + _dot_t(agg / deg, wn_ref[...])
         + (bs_ref[...] + bn_ref[...])[None, :])
    h1 = _bn_relu(t, g_ref[...], b_ref[...])
    h1f_ref[...] = h1
    h1b = h1.astype(jnp.bfloat16)
    h1s_ref[0] = h1b[:, :FH]
    h1s_ref[1] = h1b[:, FH:]


def _layer2_body(h1f_ref, p_ref, dp_ref, ws_ref, bs_ref, wn_ref, bn_ref,
                 g_ref, b_ref, wh_ref, bh_ref, out_ref):
    h1 = h1f_ref[...]
    agg = jnp.concatenate([p_ref[0], p_ref[1]], axis=1).astype(jnp.float32)
    deg = dp_ref[:, 0:1] + dp_ref[:, 1:2] + 1.0
    t = (_dot_t(h1, ws_ref[...]) + _dot_t(agg / deg, wn_ref[...])
         + (bs_ref[...] + bn_ref[...])[None, :])
    h2 = _bn_relu(t, g_ref[...], b_ref[...])
    out_ref[...] = _dot_t(h2[:N], wh_ref[...]) + bh_ref[...][None, :]


_layer1 = pl.pallas_call(
    _layer1_body, out_shape=[jax.ShapeDtypeStruct((NP, F), jnp.float32),
                             jax.ShapeDtypeStruct((NCORES, NP, FH),
                                                  jnp.bfloat16)])
_layer2 = pl.pallas_call(
    _layer2_body, out_shape=jax.ShapeDtypeStruct((N, OUT), jnp.float32))


def kernel(x, edge_index, W_self1, b_self1, W_neigh1, b_neigh1, gamma1, beta1,
           W_self2, b_self2, W_neigh2, b_neigh2, gamma2, beta2, W_head, b_head):
    x_pad = jnp.pad(x, ((0, NP - N), (0, 0)))
    xb = x_pad.astype(jnp.bfloat16)
    x2 = jnp.stack([xb[:, :FH], xb[:, FH:]])
    ei4 = edge_index.reshape(2, NSUB, NCH, ROW)
    p1, degp = _agg_with_deg(x2, ei4)
    dp_t = degp.reshape(NCORES, NP).T
    h1f, h1s = _layer1(x_pad, p1, dp_t, W_self1, b_self1, W_neigh1,
                       b_neigh1, gamma1, beta1)
    (p2,) = _agg_no_deg(h1s, ei4)
    return _layer2(h1f, p2, dp_t, W_self2, b_self2, W_neigh2, b_neigh2,
                   gamma2, beta2, W_head, b_head)
